# Initial kernel scaffold; baseline (speedup 1.0000x reference)
import jax
import jax.numpy as jnp
from jax.experimental import pallas as pl


def kernel(x, fc1_w, fc1_b, fc2_w, fc2_b, fc3_w, fc3_b, c1a_w, c1a_b, c1b_w, c1b_b, c2a_w, c2a_b, c2b_w, c2b_b, c3_w, c3_b, o1_w, o1_b, o2_w, o2_b):
    raise NotImplementedError("write your pallas kernel here")


# speedup vs baseline: 3.8167x; 3.8167x over previous
import functools

import jax
import jax.numpy as jnp
from jax.experimental import pallas as pl
from jax.experimental.pallas import tpu as pltpu

H = 128
NIN = 48
NMID = 6 * H


def _critic_body(x_ref, wa_ref, w3_ref, b_ref, o_ref):
    xb = x_ref[...].astype(jnp.bfloat16)
    y = jnp.dot(xb, wa_ref[...],
                preferred_element_type=jnp.float32) + b_ref[0:1, :]
    yb = jnp.maximum(y.astype(jnp.bfloat16), jnp.bfloat16(0.0))
    h = jnp.dot(yb, w3_ref[...],
                preferred_element_type=jnp.float32) + b_ref[1:2, 0:H]
    hr = jnp.maximum(h, 0.0)
    o_ref[...] = (jnp.sum(hr * b_ref[2:3, 0:H], axis=-1, keepdims=True)
                  + b_ref[3:4, 0:1])


def _compose_branch(a_w, b_w):
    A = a_w[:, 0, :]
    C = jnp.einsum("ck,dct->ktd", A, b_w,
                   precision=jax.lax.Precision.HIGHEST)
    M = jnp.zeros((8, H), jnp.float32)
    for t in range(5):
        M = M.at[t:t + 4, :].add(C[:, t, :])
    return M


def _prep(fc1_w, fc1_b, fc2_w, fc2_b, fc3_w, fc3_b, c1a_w, c1a_b, c1b_w,
          c1b_b, c2a_w, c2a_b, c2b_w, c2b_b, c3_w, c3_b, o1_w, o1_b,
          o2_w, o2_b):
    M1 = _compose_branch(c1a_w, c1b_w)
    M2 = _compose_branch(c2a_w, c2b_w)
    bc1 = jnp.sum(c1b_w, axis=2) @ c1a_b + c1b_b
    bc2 = jnp.sum(c2b_w, axis=2) @ c2a_b + c2b_b

    wa = jnp.zeros((NIN, NMID), jnp.float32)
    wa = wa.at[7, 0:H].set(fc1_w[:, 0])
    wa = wa.at[15, H:2 * H].set(fc2_w[:, 0])
    wa = wa.at[47, 2 * H:3 * H].set(fc3_w[:, 0])
    wa = wa.at[32:38, 3 * H:4 * H].set(c3_w[:, 0, :].T)
    wa = wa.at[16:24, 4 * H:5 * H].set(M1)
    wa = wa.at[24:32, 5 * H:6 * H].set(M2)

    ba = jnp.concatenate([fc1_b, fc2_b, fc3_b, c3_b, bc1, bc2])

    o1t = o1_w.T
    w3 = jnp.concatenate([o1t[j * H:(j + 1) * H] for j in (0, 1, 5, 4, 2, 3)],
                         axis=0)

    bias = jnp.zeros((8, NMID), jnp.float32)
    bias = bias.at[0, :].set(ba)
    bias = bias.at[1, 0:H].set(o1_b)
    bias = bias.at[2, 0:H].set(o2_w[0, :])
    bias = bias.at[3, 0].set(o2_b[0])

    return wa.astype(jnp.bfloat16), w3.astype(jnp.bfloat16), bias


@jax.jit
def kernel(x, fc1_w, fc1_b, fc2_w, fc2_b, fc3_w, fc3_b, c1a_w, c1a_b, c1b_w,
           c1b_b, c2a_w, c2a_b, c2b_w, c2b_b, c3_w, c3_b, o1_w, o1_b,
           o2_w, o2_b):
    B = x.shape[0]
    xf = x.reshape(B, NIN)
    wa, w3, bias = _prep(fc1_w, fc1_b, fc2_w, fc2_b, fc3_w, fc3_b, c1a_w,
                         c1a_b, c1b_w, c1b_b, c2a_w, c2a_b, c2b_w, c2b_b,
                         c3_w, c3_b, o1_w, o1_b, o2_w, o2_b)

    tm = 1024 if B >= 1024 else max(8, ((B + 7) // 8) * 8)
    b_pad = ((B + tm - 1) // tm) * tm
    if b_pad != B:
        xf = jnp.pad(xf, ((0, b_pad - B), (0, 0)))

    out = pl.pallas_call(
        _critic_body,
        out_shape=jax.ShapeDtypeStruct((b_pad, 1), jnp.float32),
        grid=(b_pad // tm,),
        in_specs=[
            pl.BlockSpec((tm, NIN), lambda i: (i, 0)),
            pl.BlockSpec(wa.shape, lambda i: (0, 0)),
            pl.BlockSpec(w3.shape, lambda i: (0, 0)),
            pl.BlockSpec(bias.shape, lambda i: (0, 0)),
        ],
        out_specs=pl.BlockSpec((tm, 1), lambda i: (i, 0)),
        compiler_params=pltpu.CompilerParams(
            dimension_semantics=("parallel",)),
    )(xf, wa, w3, bias)
    return out[:B]


# speedup vs baseline: 6.5766x; 1.7231x over previous
import jax
import jax.numpy as jnp
from jax.experimental import pallas as pl
from jax.experimental.pallas import tpu as pltpu

H = 128
NIN = 48
NMID = 6 * H
TN = 8192


def _critic_body(x_ref, wa_ref, w3_ref, b_ref, o_ref):
    xb = x_ref[...].astype(jnp.bfloat16)
    y = (jnp.dot(wa_ref[...], xb, preferred_element_type=jnp.float32)
         + b_ref[:, 0:1])
    yb = jnp.maximum(y.astype(jnp.bfloat16), jnp.bfloat16(0.0))
    h = (jnp.dot(w3_ref[...], yb, preferred_element_type=jnp.float32)
         + b_ref[0:H, 1:2])
    hb = jnp.maximum(h, 0.0).astype(jnp.bfloat16)
    wrow = b_ref[0:H, 2:3].astype(jnp.bfloat16).reshape(1, H)
    o = jnp.dot(wrow, hb, preferred_element_type=jnp.float32)
    o_ref[...] = o + b_ref[0:1, 3:4]


def _compose_branch_t(a_w, b_w):
    A = a_w[:, 0, :]
    C = jnp.einsum("ck,dct->ktd", A, b_w,
                   precision=jax.lax.Precision.HIGHEST)
    M = jnp.zeros((H, 8), jnp.float32)
    for t in range(5):
        M = M.at[:, t:t + 4].add(C[:, t, :].T)
    return M


def _prep(fc1_w, fc1_b, fc2_w, fc2_b, fc3_w, fc3_b, c1a_w, c1a_b, c1b_w,
          c1b_b, c2a_w, c2a_b, c2b_w, c2b_b, c3_w, c3_b, o1_w, o1_b,
          o2_w, o2_b):
    M1 = _compose_branch_t(c1a_w, c1b_w)
    M2 = _compose_branch_t(c2a_w, c2b_w)
    bc1 = jnp.sum(c1b_w, axis=2) @ c1a_b + c1b_b
    bc2 = jnp.sum(c2b_w, axis=2) @ c2a_b + c2b_b

    wa = jnp.zeros((NMID, NIN), jnp.float32)
    wa = wa.at[0:H, 7].set(fc1_w[:, 0])
    wa = wa.at[H:2 * H, 15].set(fc2_w[:, 0])
    wa = wa.at[2 * H:3 * H, 47].set(fc3_w[:, 0])
    wa = wa.at[3 * H:4 * H, 32:38].set(c3_w[:, 0, :])
    wa = wa.at[4 * H:5 * H, 16:24].set(M1)
    wa = wa.at[5 * H:6 * H, 24:32].set(M2)

    ba = jnp.concatenate([fc1_b, fc2_b, fc3_b, c3_b, bc1, bc2])

    w3 = jnp.concatenate([o1_w[:, j * H:(j + 1) * H]
                          for j in (0, 1, 5, 4, 2, 3)], axis=1)

    bias = jnp.zeros((NMID, 8), jnp.float32)
    bias = bias.at[:, 0].set(ba)
    bias = bias.at[0:H, 1].set(o1_b)
    bias = bias.at[0:H, 2].set(o2_w[0, :])
    bias = bias.at[0, 3].set(o2_b[0])

    return wa.astype(jnp.bfloat16), w3.astype(jnp.bfloat16), bias


@jax.jit
def kernel(x, fc1_w, fc1_b, fc2_w, fc2_b, fc3_w, fc3_b, c1a_w, c1a_b, c1b_w,
           c1b_b, c2a_w, c2a_b, c2b_w, c2b_b, c3_w, c3_b, o1_w, o1_b,
           o2_w, o2_b):
    B = x.shape[0]
    wa, w3, bias = _prep(fc1_w, fc1_b, fc2_w, fc2_b, fc3_w, fc3_b, c1a_w,
                         c1a_b, c1b_w, c1b_b, c2a_w, c2a_b, c2b_w, c2b_b,
                         c3_w, c3_b, o1_w, o1_b, o2_w, o2_b)

    xT = x.reshape(B, NIN).T
    tn = TN if B % TN == 0 else 128
    b_pad = ((B + tn - 1) // tn) * tn
    if b_pad != B:
        xT = jnp.pad(xT, ((0, 0), (0, b_pad - B)))

    out = pl.pallas_call(
        _critic_body,
        out_shape=jax.ShapeDtypeStruct((1, b_pad), jnp.float32),
        grid=(b_pad // tn,),
        in_specs=[
            pl.BlockSpec((NIN, tn), lambda i: (0, i)),
            pl.BlockSpec(wa.shape, lambda i: (0, 0)),
            pl.BlockSpec(w3.shape, lambda i: (0, 0)),
            pl.BlockSpec(bias.shape, lambda i: (0, 0)),
        ],
        out_specs=pl.BlockSpec((1, tn), lambda i: (0, i)),
        compiler_params=pltpu.CompilerParams(
            dimension_semantics=("arbitrary",)),
    )(xT, wa, w3, bias)
    return out[:, :B].reshape(B, 1)
